# baseline (speedup 1.0000x reference)
import functools

import jax
import jax.numpy as jnp
from jax import lax
from jax.experimental import pallas as pl
from jax.experimental.pallas import tpu as pltpu
from jax.experimental.pallas import tpu_sc as plsc


_DIM = 32
_BATCH = 16384


@functools.lru_cache(maxsize=None)
def _build(V, D, B):
    info = plsc.get_sparse_core_info()
    NW = info.num_cores * info.num_subcores
    assert B % NW == 0
    b_per_w = B // NW
    G = V // 8
    mesh = plsc.VectorSubcoreMesh(core_axis_name="c", subcore_axis_name="s")

    @functools.partial(
        pl.kernel,
        mesh=mesh,
        out_type=jax.ShapeDtypeStruct((B, D), jnp.float32),
        scratch_types=[
            pltpu.VMEM((b_per_w,), jnp.int32),
            pltpu.VMEM((b_per_w,), jnp.int32),
            pltpu.VMEM((b_per_w, D), jnp.float32),
            pltpu.SemaphoreType.DMA,
        ],
        compiler_params=pltpu.CompilerParams(disable_bounds_checks=True),
    )
    def gather_kernel(idx_hbm, table_hbm, out_hbm, g_v, s_v, rows_v, sem):
        wid = lax.axis_index("s") * info.num_cores + lax.axis_index("c")
        base = wid * b_per_w
        pltpu.sync_copy(idx_hbm.at[pl.ds(base, b_per_w)], g_v)

        for k in range(b_per_w // 16):
            tv = g_v[pl.ds(k * 16, 16)]
            g = jnp.minimum(tv >> 3, G - 1)
            s = tv - (g << 3)
            g_v[pl.ds(k * 16, 16)] = g
            s_v[pl.ds(k * 16, 16)] = s

        tbl3d = table_hbm.at[pl.ds(0, G * 8), :].reshape(G, 8, D)

        copies = []
        for k in range(b_per_w // 16):
            gv = g_v[pl.ds(k * 16, 16)]
            sv = s_v[pl.ds(k * 16, 16)]
            for j in range(16):
                i = k * 16 + j
                copies.append(
                    pltpu.async_copy(
                        tbl3d.at[gv[j], pl.ds(sv[j], 1), :],
                        rows_v.at[pl.ds(i, 1), :],
                        sem,
                    )
                )
        for c in copies:
            c.wait()
        pltpu.sync_copy(rows_v, out_hbm.at[pl.ds(base, b_per_w)])

    return gather_kernel


def kernel(t, emb_weight):
    fn = _build(emb_weight.shape[0], _DIM, _BATCH)
    return fn(t.astype(jnp.int32), emb_weight)


# speedup vs baseline: 1.0101x; 1.0101x over previous
import functools

import jax
import jax.numpy as jnp
from jax import lax
from jax.experimental import pallas as pl
from jax.experimental.pallas import tpu as pltpu
from jax.experimental.pallas import tpu_sc as plsc


_DIM = 32
_BATCH = 16384


@functools.lru_cache(maxsize=None)
def _build(V, D, B):
    info = plsc.get_sparse_core_info()
    NC, NS = info.num_cores, info.num_subcores
    NW = NC * NS
    assert B % NW == 0
    b_per_w = B // NW
    half = B // NC
    mesh = plsc.VectorSubcoreMesh(core_axis_name="c", subcore_axis_name="s")

    @functools.partial(
        pl.kernel,
        mesh=mesh,
        out_type=(
            jax.ShapeDtypeStruct((half, D), jnp.float32),
            jax.ShapeDtypeStruct((half, D), jnp.float32),
        ),
        scratch_types=[
            pltpu.VMEM((b_per_w,), jnp.int32),
            pltpu.VMEM((b_per_w, D), jnp.float32),
            pltpu.SemaphoreType.DMA,
        ],
    )
    def gather_kernel(idx_hbm, table_hbm, out0_hbm, out1_hbm, t_v, rows_v,
                      sem):
        cid = lax.axis_index("c")
        sid = lax.axis_index("s")
        base = (cid * NS + sid) * b_per_w
        lbase = sid * b_per_w
        pltpu.sync_copy(idx_hbm.at[pl.ds(base, b_per_w)], t_v)

        copies = []
        for k in range(b_per_w // 16):
            tv = t_v[pl.ds(k * 16, 16)]
            for j in range(16):
                copies.append(
                    pltpu.async_copy(
                        table_hbm.at[pl.ds(tv[j], 1), :],
                        rows_v.at[pl.ds(k * 16 + j, 1), :],
                        sem,
                    )
                )
        for c in copies:
            c.wait()

        @pl.when(cid == 0)
        def _():
            pltpu.sync_copy(rows_v, out0_hbm.at[pl.ds(lbase, b_per_w)])

        @pl.when(cid == 1)
        def _():
            pltpu.sync_copy(rows_v, out1_hbm.at[pl.ds(lbase, b_per_w)])

    return gather_kernel


def kernel(t, emb_weight):
    fn = _build(emb_weight.shape[0], _DIM, _BATCH)
    out0, out1 = fn(t.astype(jnp.int32), emb_weight)
    return jnp.concatenate([out0, out1], axis=0)


# speedup vs baseline: 1.0121x; 1.0021x over previous
import functools

import jax
import jax.numpy as jnp
from jax import lax
from jax.experimental import pallas as pl
from jax.experimental.pallas import tpu as pltpu
from jax.experimental.pallas import tpu_sc as plsc


_DIM = 32
_BATCH = 16384


@functools.lru_cache(maxsize=None)
def _build(V, D, B):
    info = plsc.get_sparse_core_info()
    NW = info.num_cores * info.num_subcores
    assert B % NW == 0
    b_per_w = B // NW
    mesh = plsc.VectorSubcoreMesh(core_axis_name="c", subcore_axis_name="s")

    @functools.partial(
        pl.kernel,
        mesh=mesh,
        out_type=jax.ShapeDtypeStruct((B, D), jnp.float32),
        scratch_types=[
            pltpu.VMEM((b_per_w,), jnp.int32),
            pltpu.VMEM((b_per_w, D), jnp.float32),
            pltpu.SemaphoreType.DMA,
        ],
    )
    def gather_kernel(idx_hbm, table_hbm, out_hbm, t_v, rows_v, sem):
        wid = lax.axis_index("s") * info.num_cores + lax.axis_index("c")
        base = wid * b_per_w
        pltpu.sync_copy(idx_hbm.at[pl.ds(base, b_per_w)], t_v)

        copies = []
        for k in range(b_per_w // 16):
            tv = t_v[pl.ds(k * 16, 16)]
            for j in range(16):
                copies.append(
                    pltpu.async_copy(
                        table_hbm.at[pl.ds(tv[j], 1), :],
                        rows_v.at[pl.ds(k * 16 + j, 1), :],
                        sem,
                    )
                )
        for c in copies:
            c.wait()
        pltpu.sync_copy(rows_v, out_hbm.at[pl.ds(base, b_per_w)])

    return gather_kernel


def kernel(t, emb_weight):
    fn = _build(emb_weight.shape[0], _DIM, _BATCH)
    return fn(t.astype(jnp.int32), emb_weight)


# speedup vs baseline: 1.0125x; 1.0003x over previous
import functools

import jax
import jax.numpy as jnp
from jax import lax
from jax.experimental import pallas as pl
from jax.experimental.pallas import tpu as pltpu
from jax.experimental.pallas import tpu_sc as plsc


_DIM = 32
_BATCH = 16384


@functools.lru_cache(maxsize=None)
def _build(V, D, B):
    info = plsc.get_sparse_core_info()
    NW = info.num_cores * info.num_subcores
    assert B % NW == 0
    b_per_w = B // NW
    mesh = plsc.VectorSubcoreMesh(core_axis_name="c", subcore_axis_name="s")

    @functools.partial(
        pl.kernel,
        mesh=mesh,
        out_type=jax.ShapeDtypeStruct((B, D), jnp.float32),
        scratch_types=[
            pltpu.VMEM((b_per_w,), jnp.int32),
            pltpu.VMEM((b_per_w, D), jnp.float32),
            pltpu.SemaphoreType.DMA,
        ],
        compiler_params=pltpu.CompilerParams(skip_device_barrier=True),
    )
    def gather_kernel(idx_hbm, table_hbm, out_hbm, t_v, rows_v, sem):
        wid = lax.axis_index("s") * info.num_cores + lax.axis_index("c")
        base = wid * b_per_w
        pltpu.sync_copy(idx_hbm.at[pl.ds(base, b_per_w)], t_v)

        copies = []
        for k in range(b_per_w // 16):
            tv = t_v[pl.ds(k * 16, 16)]
            for j in range(16):
                copies.append(
                    pltpu.async_copy(
                        table_hbm.at[pl.ds(tv[j], 1), :],
                        rows_v.at[pl.ds(k * 16 + j, 1), :],
                        sem,
                    )
                )
        for c in copies:
            c.wait()
        pltpu.sync_copy(rows_v, out_hbm.at[pl.ds(base, b_per_w)])

    return gather_kernel


def kernel(t, emb_weight):
    fn = _build(emb_weight.shape[0], _DIM, _BATCH)
    return fn(t.astype(jnp.int32), emb_weight)
